# baseline (speedup 1.0000x reference)
import functools

import jax
import jax.numpy as jnp
from jax import lax
from jax.experimental import pallas as pl
from jax.experimental.pallas import tpu as pltpu
from jax.experimental.pallas import tpu_sc as plsc

N_EMBED = 1024
NUM_EXPERTS = 16
TOP_K = 2
D_FF = 4 * N_EMBED
T_TOKENS = 4096
EPAD = 128
BM = 256
NB = 48
NSLOT = NB * BM
NW = 32
TPW = T_TOKENS // NW
CHUNK = 32
NCH = TPW // CHUNK


def _router_body(x_ref, wg_ref, bg_ref, wn_ref, bn_ref, nz_ref,
                 pos0_ref, pos1_ref, p0_ref, p1_ref, be_ref, act_ref):
    f32 = jnp.float32
    x = x_ref[...]
    logits = jnp.dot(x, wg_ref[...], preferred_element_type=f32) + bg_ref[...]
    nlog = jnp.dot(x, wn_ref[...], preferred_element_type=f32) + bn_ref[...]
    sp = jnp.maximum(nlog, 0.0) + jnp.log1p(jnp.exp(-jnp.abs(nlog)))
    noisy = logits + nz_ref[...] * sp

    idx = lax.broadcasted_iota(jnp.int32, (T_TOKENS, EPAD), 1)
    v1 = jnp.max(noisy, axis=1, keepdims=True)
    i1 = jnp.min(jnp.where(noisy == v1, idx, EPAD - 1), axis=1, keepdims=True)
    m0 = idx == i1
    masked = jnp.where(m0, -1e30, noisy)
    v2 = jnp.max(masked, axis=1, keepdims=True)
    i2 = jnp.min(jnp.where(masked == v2, idx, EPAD - 1), axis=1, keepdims=True)
    m1 = idx == i2
    d = jnp.exp(v2 - v1)
    p0_ref[...] = 1.0 / (1.0 + d)
    p1_ref[...] = d / (1.0 + d)

    m0f = m0.astype(f32)
    m1f = m1.astype(f32)

    blk = 512
    r = lax.broadcasted_iota(jnp.int32, (blk, blk), 0)
    c = lax.broadcasted_iota(jnp.int32, (blk, blk), 1)
    ls = (r > c).astype(f32)
    carry = jnp.zeros((1, EPAD), f32)
    ranks = []
    for m in (m0f, m1f):
        for b in range(T_TOKENS // blk):
            mb = m[b * blk:(b + 1) * blk]
            s = jnp.dot(ls, mb, preferred_element_type=f32) + carry
            ranks.append(jnp.sum(s * mb, axis=1, keepdims=True))
            carry = carry + jnp.sum(mb, axis=0, keepdims=True)
    rank0 = jnp.concatenate(ranks[:T_TOKENS // blk], axis=0)
    rank1 = jnp.concatenate(ranks[T_TOKENS // blk:], axis=0)

    counts = carry
    pc = jnp.floor((counts + (BM - 1)) / BM) * BM
    rr = lax.broadcasted_iota(jnp.int32, (EPAD, EPAD), 0)
    cc = lax.broadcasted_iota(jnp.int32, (EPAD, EPAD), 1)
    tu = (rr < cc).astype(f32)
    offs = jnp.dot(pc, tu, preferred_element_type=f32)

    pos0 = rank0 + jnp.sum(m0f * offs, axis=1, keepdims=True)
    pos1 = rank1 + jnp.sum(m1f * offs, axis=1, keepdims=True)
    pos0_ref[...] = pos0.astype(jnp.int32)
    pos1_ref[...] = pos1.astype(jnp.int32)

    pend = offs + pc
    bstart = (lax.broadcasted_iota(jnp.int32, (NB, 1), 0) * BM).astype(f32)
    lane_ok = lax.broadcasted_iota(jnp.int32, (1, EPAD), 1) < NUM_EXPERTS
    done = jnp.where(lane_ok & (pend <= bstart), 1, 0)
    nd = jnp.sum(done, axis=1, keepdims=True)
    be_ref[...] = jnp.minimum(nd, NUM_EXPERTS - 1).astype(jnp.int32)
    act_ref[...] = jnp.where(nd < NUM_EXPERTS, 1, 0).astype(jnp.int32)


def _run_router(xr, wg_p, bg_p, wn_p, bn_p, noise_p):
    f32 = jnp.float32
    return pl.pallas_call(
        _router_body,
        out_shape=[
            jax.ShapeDtypeStruct((T_TOKENS, 1), jnp.int32),
            jax.ShapeDtypeStruct((T_TOKENS, 1), jnp.int32),
            jax.ShapeDtypeStruct((T_TOKENS, 1), f32),
            jax.ShapeDtypeStruct((T_TOKENS, 1), f32),
            jax.ShapeDtypeStruct((NB, 1), jnp.int32),
            jax.ShapeDtypeStruct((NB, 1), jnp.int32),
        ],
        compiler_params=pltpu.CompilerParams(
            vmem_limit_bytes=100 * 1024 * 1024),
    )(xr, wg_p, bg_p, wn_p, bn_p, noise_p)


def _dispatch_body(x_hbm, idx_hbm, xs_hbm, idx_v, xa_v, sem):
    wid = lax.axis_index("s") * 2 + lax.axis_index("c")
    pltpu.sync_copy(idx_hbm.at[wid], idx_v)
    base = wid * TPW
    for ch in range(NCH):
        pltpu.sync_copy(x_hbm.at[pl.ds(base + ch * CHUNK, CHUNK)], xa_v)
        pltpu.async_copy(xa_v, xs_hbm.at[idx_v.at[ch]], sem).wait()
        pltpu.async_copy(xa_v, xs_hbm.at[idx_v.at[NCH + ch]], sem).wait()


def _run_dispatch(xr, idx_all):
    mesh = plsc.VectorSubcoreMesh(core_axis_name="c", subcore_axis_name="s")
    return pl.kernel(
        _dispatch_body,
        mesh=mesh,
        out_type=jax.ShapeDtypeStruct((NSLOT, N_EMBED), jnp.float32),
        scratch_types=[
            pltpu.VMEM((2 * NCH, CHUNK), jnp.int32),
            pltpu.VMEM((CHUNK, N_EMBED), jnp.float32),
            pltpu.SemaphoreType.DMA,
        ],
    )(xr, idx_all)


def _gemm_body(be_ref, act_ref, x_ref, b1_ref, b2_ref, w1_hbm, w2_hbm,
               o_ref, w1v, w2v, sem1, sem2):
    f32 = jnp.float32
    b = pl.program_id(0)
    cur = be_ref[b]
    prev = be_ref[jnp.maximum(b - 1, 0)]

    @pl.when((b == 0) | (b == NB // 2) | (cur != prev))
    def _():
        cp1 = pltpu.make_async_copy(w1_hbm.at[cur], w1v, sem1)
        cp2 = pltpu.make_async_copy(w2_hbm.at[cur], w2v, sem2)
        cp1.start()
        cp2.start()
        cp1.wait()
        cp2.wait()

    @pl.when(act_ref[b] == 1)
    def _():
        x = x_ref[...]
        h = jnp.dot(x, w1v[...], preferred_element_type=f32) + b1_ref[cur]
        h = jnp.maximum(h, 0.0)
        o_ref[...] = jnp.dot(h, w2v[...], preferred_element_type=f32) \
            + b2_ref[cur]


def _run_gemm(xs, w1, b1r, w2, b2r, be, act):
    f32 = jnp.float32
    grid_spec = pltpu.PrefetchScalarGridSpec(
        num_scalar_prefetch=2,
        grid=(NB,),
        in_specs=[
            pl.BlockSpec((BM, N_EMBED), lambda b, be, act: (b, 0)),
            pl.BlockSpec((NUM_EXPERTS, 1, D_FF),
                         lambda b, be, act: (0, 0, 0)),
            pl.BlockSpec((NUM_EXPERTS, 1, N_EMBED),
                         lambda b, be, act: (0, 0, 0)),
            pl.BlockSpec(memory_space=pl.ANY),
            pl.BlockSpec(memory_space=pl.ANY),
        ],
        out_specs=pl.BlockSpec((BM, N_EMBED), lambda b, be, act: (b, 0)),
        scratch_shapes=[
            pltpu.VMEM((N_EMBED, D_FF), f32),
            pltpu.VMEM((D_FF, N_EMBED), f32),
            pltpu.SemaphoreType.DMA,
            pltpu.SemaphoreType.DMA,
        ],
    )
    return pl.pallas_call(
        _gemm_body,
        grid_spec=grid_spec,
        out_shape=jax.ShapeDtypeStruct((NSLOT, N_EMBED), jnp.float32),
        compiler_params=pltpu.CompilerParams(
            dimension_semantics=("parallel",),
            vmem_limit_bytes=60 * 1024 * 1024),
    )(be, act, xs, b1r, b2r, w1, w2)


def _combine_body(ys_hbm, idx_hbm, p_hbm, out_hbm,
                  idx_v, p_v, buf0, buf1, sem):
    wid = lax.axis_index("s") * 2 + lax.axis_index("c")
    pltpu.sync_copy(idx_hbm.at[wid], idx_v)
    pltpu.sync_copy(p_hbm.at[wid], p_v)
    base = wid * TPW
    lanes = N_EMBED // 16
    for ch in range(NCH):
        pltpu.async_copy(ys_hbm.at[idx_v.at[ch]], buf0, sem).wait()
        pltpu.async_copy(ys_hbm.at[idx_v.at[NCH + ch]], buf1, sem).wait()
        for j in range(CHUNK):
            half = pl.ds((j // 16) * 16, 16)
            pa = p_v[ch, half]
            pb = p_v[NCH + ch, half]
            sel = jnp.full((16, 1), j % 16, jnp.int32)
            dnums = lax.GatherDimensionNumbers(
                offset_dims=(), collapsed_slice_dims=(0,),
                start_index_map=(0,))
            w0 = lax.gather(pa, sel, dnums, (1,),
                            mode=lax.GatherScatterMode.PROMISE_IN_BOUNDS)
            w1 = lax.gather(pb, sel, dnums, (1,),
                            mode=lax.GatherScatterMode.PROMISE_IN_BOUNDS)

            def body(v, _):
                sl = pl.ds(v * 16, 16)
                buf0[j, sl] = buf0[j, sl] * w0 + buf1[j, sl] * w1
                return _

            lax.fori_loop(0, lanes, body, 0, unroll=4)
        pltpu.sync_copy(buf0, out_hbm.at[pl.ds(base + ch * CHUNK, CHUNK)])


def _run_combine(ys, idx_all, p_all):
    mesh = plsc.VectorSubcoreMesh(core_axis_name="c", subcore_axis_name="s")
    return pl.kernel(
        _combine_body,
        mesh=mesh,
        out_type=jax.ShapeDtypeStruct((T_TOKENS, N_EMBED), jnp.float32),
        scratch_types=[
            pltpu.VMEM((2 * NCH, CHUNK), jnp.int32),
            pltpu.VMEM((2 * NCH, CHUNK), jnp.float32),
            pltpu.VMEM((CHUNK, N_EMBED), jnp.float32),
            pltpu.VMEM((CHUNK, N_EMBED), jnp.float32),
            pltpu.SemaphoreType.DMA,
        ],
    )(ys, idx_all, p_all)


def kernel(x, Wg, bg, Wn, bn, W1, b1, W2, b2):
    f32 = jnp.float32
    B, S, D = x.shape
    xr = x.reshape(B * S, D)

    wg_p = jnp.pad(Wg, ((0, 0), (0, EPAD - NUM_EXPERTS)))
    wn_p = jnp.pad(Wn, ((0, 0), (0, EPAD - NUM_EXPERTS)))
    bg_p = jnp.pad(bg, (0, EPAD - NUM_EXPERTS),
                   constant_values=-1e30).reshape(1, EPAD)
    bn_p = jnp.pad(bn, (0, EPAD - NUM_EXPERTS)).reshape(1, EPAD)
    noise = jax.random.normal(jax.random.key(42), (B * S, NUM_EXPERTS),
                              dtype=f32)
    noise_p = jnp.pad(noise, ((0, 0), (0, EPAD - NUM_EXPERTS)))

    pos0, pos1, p0, p1, be, act = _run_router(xr, wg_p, bg_p, wn_p, bn_p,
                                              noise_p)

    idx_all = jnp.concatenate(
        [pos0.reshape(NW, NCH, CHUNK), pos1.reshape(NW, NCH, CHUNK)], axis=1)
    p_all = jnp.concatenate(
        [p0.reshape(NW, NCH, CHUNK), p1.reshape(NW, NCH, CHUNK)], axis=1)

    xs = _run_dispatch(xr, idx_all)
    ys = _run_gemm(xs, W1, b1.reshape(NUM_EXPERTS, 1, D_FF), W2,
                   b2.reshape(NUM_EXPERTS, 1, N_EMBED), be.reshape(NB),
                   act.reshape(NB))
    out = _run_combine(ys, idx_all, p_all)
    return out.reshape(B, S, D)


# speedup vs baseline: 1.0305x; 1.0305x over previous
import functools

import jax
import jax.numpy as jnp
from jax import lax
from jax.experimental import pallas as pl
from jax.experimental.pallas import tpu as pltpu
from jax.experimental.pallas import tpu_sc as plsc

N_EMBED = 1024
NUM_EXPERTS = 16
TOP_K = 2
D_FF = 4 * N_EMBED
T_TOKENS = 4096
EPAD = 128
BM = 256
NB = 48
NSLOT = NB * BM
NW = 32
TPW = T_TOKENS // NW
CHUNK = 32
NCH = TPW // CHUNK


def _router_body(x_ref, wg_ref, bg_ref, wn_ref, bn_ref, nz_ref,
                 pos0_ref, pos1_ref, p0_ref, p1_ref, be_ref, act_ref):
    f32 = jnp.float32
    x = x_ref[...]
    logits = jnp.dot(x, wg_ref[...], preferred_element_type=f32) + bg_ref[...]
    nlog = jnp.dot(x, wn_ref[...], preferred_element_type=f32) + bn_ref[...]
    sp = jnp.maximum(nlog, 0.0) + jnp.log1p(jnp.exp(-jnp.abs(nlog)))
    noisy = logits + nz_ref[...] * sp

    idx = lax.broadcasted_iota(jnp.int32, (T_TOKENS, EPAD), 1)
    v1 = jnp.max(noisy, axis=1, keepdims=True)
    i1 = jnp.min(jnp.where(noisy == v1, idx, EPAD - 1), axis=1, keepdims=True)
    m0 = idx == i1
    masked = jnp.where(m0, -1e30, noisy)
    v2 = jnp.max(masked, axis=1, keepdims=True)
    i2 = jnp.min(jnp.where(masked == v2, idx, EPAD - 1), axis=1, keepdims=True)
    m1 = idx == i2
    d = jnp.exp(v2 - v1)
    p0_ref[...] = 1.0 / (1.0 + d)
    p1_ref[...] = d / (1.0 + d)

    m0f = m0.astype(f32)
    m1f = m1.astype(f32)

    blk = 512
    r = lax.broadcasted_iota(jnp.int32, (blk, blk), 0)
    c = lax.broadcasted_iota(jnp.int32, (blk, blk), 1)
    ls = (r > c).astype(f32)
    carry = jnp.zeros((1, EPAD), f32)
    ranks = []
    for m in (m0f, m1f):
        for b in range(T_TOKENS // blk):
            mb = m[b * blk:(b + 1) * blk]
            s = jnp.dot(ls, mb, preferred_element_type=f32) + carry
            ranks.append(jnp.sum(s * mb, axis=1, keepdims=True))
            carry = carry + jnp.sum(mb, axis=0, keepdims=True)
    rank0 = jnp.concatenate(ranks[:T_TOKENS // blk], axis=0)
    rank1 = jnp.concatenate(ranks[T_TOKENS // blk:], axis=0)

    counts = carry
    pc = jnp.floor((counts + (BM - 1)) / BM) * BM
    rr = lax.broadcasted_iota(jnp.int32, (EPAD, EPAD), 0)
    cc = lax.broadcasted_iota(jnp.int32, (EPAD, EPAD), 1)
    tu = (rr < cc).astype(f32)
    offs = jnp.dot(pc, tu, preferred_element_type=f32)

    pos0 = rank0 + jnp.sum(m0f * offs, axis=1, keepdims=True)
    pos1 = rank1 + jnp.sum(m1f * offs, axis=1, keepdims=True)
    pos0_ref[...] = pos0.astype(jnp.int32)
    pos1_ref[...] = pos1.astype(jnp.int32)

    pend = offs + pc
    bstart = (lax.broadcasted_iota(jnp.int32, (NB, 1), 0) * BM).astype(f32)
    lane_ok = lax.broadcasted_iota(jnp.int32, (1, EPAD), 1) < NUM_EXPERTS
    done = jnp.where(lane_ok & (pend <= bstart), 1, 0)
    nd = jnp.sum(done, axis=1, keepdims=True)
    be_ref[...] = jnp.minimum(nd, NUM_EXPERTS - 1).astype(jnp.int32)
    act_ref[...] = jnp.where(nd < NUM_EXPERTS, 1, 0).astype(jnp.int32)


def _run_router(xr, wg_p, bg_p, wn_p, bn_p, noise_p):
    f32 = jnp.float32
    return pl.pallas_call(
        _router_body,
        out_shape=[
            jax.ShapeDtypeStruct((T_TOKENS, 1), jnp.int32),
            jax.ShapeDtypeStruct((T_TOKENS, 1), jnp.int32),
            jax.ShapeDtypeStruct((T_TOKENS, 1), f32),
            jax.ShapeDtypeStruct((T_TOKENS, 1), f32),
            jax.ShapeDtypeStruct((NB, 1), jnp.int32),
            jax.ShapeDtypeStruct((NB, 1), jnp.int32),
        ],
        compiler_params=pltpu.CompilerParams(
            vmem_limit_bytes=100 * 1024 * 1024),
    )(xr, wg_p, bg_p, wn_p, bn_p, noise_p)


def _dispatch_body(x_hbm, idx_hbm, xs_hbm, idx_v, xa_v, sem):
    wid = lax.axis_index("s") * 2 + lax.axis_index("c")
    pltpu.sync_copy(idx_hbm.at[wid], idx_v)
    base = wid * TPW
    for ch in range(NCH):
        pltpu.sync_copy(x_hbm.at[pl.ds(base + ch * CHUNK, CHUNK)], xa_v)
        pltpu.async_copy(xa_v, xs_hbm.at[idx_v.at[ch]], sem).wait()
        pltpu.async_copy(xa_v, xs_hbm.at[idx_v.at[NCH + ch]], sem).wait()


def _run_dispatch(xr, idx_all):
    mesh = plsc.VectorSubcoreMesh(core_axis_name="c", subcore_axis_name="s")
    return pl.kernel(
        _dispatch_body,
        mesh=mesh,
        out_type=jax.ShapeDtypeStruct((NSLOT, N_EMBED), jnp.float32),
        scratch_types=[
            pltpu.VMEM((2 * NCH, CHUNK), jnp.int32),
            pltpu.VMEM((CHUNK, N_EMBED), jnp.float32),
            pltpu.SemaphoreType.DMA,
        ],
    )(xr, idx_all)


def _gemm_body(be_ref, act_ref, x_ref, b1_ref, b2_ref, w1_hbm, w2_hbm,
               o_ref, w1v, w2v, sem1, sem2):
    f32 = jnp.float32
    b = pl.program_id(0)
    cur = be_ref[b]
    prev = be_ref[jnp.maximum(b - 1, 0)]

    @pl.when(b == 0)
    def _():
        pltpu.make_async_copy(w1_hbm.at[cur], w1v, sem1).start()
        pltpu.make_async_copy(w2_hbm.at[cur], w2v, sem2).start()

    @pl.when(jnp.logical_or(b == 0, cur != prev))
    def _():
        pltpu.make_async_copy(w1_hbm.at[cur], w1v, sem1).wait()
        pltpu.make_async_copy(w2_hbm.at[cur], w2v, sem2).wait()

    @pl.when(act_ref[b] == 1)
    def _():
        x = x_ref[...]
        h = jnp.dot(x, w1v[...], preferred_element_type=f32) + b1_ref[cur]
        h = jnp.maximum(h, 0.0)
        o_ref[...] = jnp.dot(h, w2v[...], preferred_element_type=f32) \
            + b2_ref[cur]

    nxt = be_ref[jnp.minimum(b + 1, NB - 1)]

    @pl.when(jnp.logical_and(b + 1 < NB, nxt != cur))
    def _():
        pltpu.make_async_copy(w1_hbm.at[nxt], w1v, sem1).start()
        pltpu.make_async_copy(w2_hbm.at[nxt], w2v, sem2).start()


def _run_gemm(xs, w1, b1r, w2, b2r, be, act):
    f32 = jnp.float32
    grid_spec = pltpu.PrefetchScalarGridSpec(
        num_scalar_prefetch=2,
        grid=(NB,),
        in_specs=[
            pl.BlockSpec((BM, N_EMBED), lambda b, be, act: (b, 0)),
            pl.BlockSpec((NUM_EXPERTS, 1, D_FF),
                         lambda b, be, act: (0, 0, 0)),
            pl.BlockSpec((NUM_EXPERTS, 1, N_EMBED),
                         lambda b, be, act: (0, 0, 0)),
            pl.BlockSpec(memory_space=pl.ANY),
            pl.BlockSpec(memory_space=pl.ANY),
        ],
        out_specs=pl.BlockSpec((BM, N_EMBED), lambda b, be, act: (b, 0)),
        scratch_shapes=[
            pltpu.VMEM((N_EMBED, D_FF), f32),
            pltpu.VMEM((D_FF, N_EMBED), f32),
            pltpu.SemaphoreType.DMA,
            pltpu.SemaphoreType.DMA,
        ],
    )
    return pl.pallas_call(
        _gemm_body,
        grid_spec=grid_spec,
        out_shape=jax.ShapeDtypeStruct((NSLOT, N_EMBED), jnp.float32),
        compiler_params=pltpu.CompilerParams(
            vmem_limit_bytes=60 * 1024 * 1024),
    )(be, act, xs, b1r, b2r, w1, w2)


def _combine_body(ys_hbm, idx_hbm, p_hbm, out_hbm,
                  idx_v, p_v, buf0, buf1, sem):
    wid = lax.axis_index("s") * 2 + lax.axis_index("c")
    pltpu.sync_copy(idx_hbm.at[wid], idx_v)
    pltpu.sync_copy(p_hbm.at[wid], p_v)
    base = wid * TPW
    lanes = N_EMBED // 16
    for ch in range(NCH):
        pltpu.async_copy(ys_hbm.at[idx_v.at[ch]], buf0, sem).wait()
        pltpu.async_copy(ys_hbm.at[idx_v.at[NCH + ch]], buf1, sem).wait()
        for j in range(CHUNK):
            half = pl.ds((j // 16) * 16, 16)
            pa = p_v[ch, half]
            pb = p_v[NCH + ch, half]
            sel = jnp.full((16, 1), j % 16, jnp.int32)
            dnums = lax.GatherDimensionNumbers(
                offset_dims=(), collapsed_slice_dims=(0,),
                start_index_map=(0,))
            w0 = lax.gather(pa, sel, dnums, (1,),
                            mode=lax.GatherScatterMode.PROMISE_IN_BOUNDS)
            w1 = lax.gather(pb, sel, dnums, (1,),
                            mode=lax.GatherScatterMode.PROMISE_IN_BOUNDS)

            def body(v, _):
                sl = pl.ds(v * 16, 16)
                buf0[j, sl] = buf0[j, sl] * w0 + buf1[j, sl] * w1
                return _

            lax.fori_loop(0, lanes, body, 0, unroll=4)
        pltpu.sync_copy(buf0, out_hbm.at[pl.ds(base + ch * CHUNK, CHUNK)])


def _run_combine(ys, idx_all, p_all):
    mesh = plsc.VectorSubcoreMesh(core_axis_name="c", subcore_axis_name="s")
    return pl.kernel(
        _combine_body,
        mesh=mesh,
        out_type=jax.ShapeDtypeStruct((T_TOKENS, N_EMBED), jnp.float32),
        scratch_types=[
            pltpu.VMEM((2 * NCH, CHUNK), jnp.int32),
            pltpu.VMEM((2 * NCH, CHUNK), jnp.float32),
            pltpu.VMEM((CHUNK, N_EMBED), jnp.float32),
            pltpu.VMEM((CHUNK, N_EMBED), jnp.float32),
            pltpu.SemaphoreType.DMA,
        ],
    )(ys, idx_all, p_all)


def kernel(x, Wg, bg, Wn, bn, W1, b1, W2, b2):
    f32 = jnp.float32
    B, S, D = x.shape
    xr = x.reshape(B * S, D)

    wg_p = jnp.pad(Wg, ((0, 0), (0, EPAD - NUM_EXPERTS)))
    wn_p = jnp.pad(Wn, ((0, 0), (0, EPAD - NUM_EXPERTS)))
    bg_p = jnp.pad(bg, (0, EPAD - NUM_EXPERTS),
                   constant_values=-1e30).reshape(1, EPAD)
    bn_p = jnp.pad(bn, (0, EPAD - NUM_EXPERTS)).reshape(1, EPAD)
    noise = jax.random.normal(jax.random.key(42), (B * S, NUM_EXPERTS),
                              dtype=f32)
    noise_p = jnp.pad(noise, ((0, 0), (0, EPAD - NUM_EXPERTS)))

    pos0, pos1, p0, p1, be, act = _run_router(xr, wg_p, bg_p, wn_p, bn_p,
                                              noise_p)

    idx_all = jnp.concatenate(
        [pos0.reshape(NW, NCH, CHUNK), pos1.reshape(NW, NCH, CHUNK)], axis=1)
    p_all = jnp.concatenate(
        [p0.reshape(NW, NCH, CHUNK), p1.reshape(NW, NCH, CHUNK)], axis=1)

    xs = _run_dispatch(xr, idx_all)
    ys = _run_gemm(xs, W1, b1.reshape(NUM_EXPERTS, 1, D_FF), W2,
                   b2.reshape(NUM_EXPERTS, 1, N_EMBED), be.reshape(NB),
                   act.reshape(NB))
    out = _run_combine(ys, idx_all, p_all)
    return out.reshape(B, S, D)


# speedup vs baseline: 1.2680x; 1.2305x over previous
import functools

import jax
import jax.numpy as jnp
from jax import lax
from jax.experimental import pallas as pl
from jax.experimental.pallas import tpu as pltpu
from jax.experimental.pallas import tpu_sc as plsc

N_EMBED = 1024
NUM_EXPERTS = 16
TOP_K = 2
D_FF = 4 * N_EMBED
T_TOKENS = 4096
EPAD = 128
BM = 256
NB = 48
NSLOT = NB * BM
NW = 32
TPW = T_TOKENS // NW
CHUNK = 32
NCH = TPW // CHUNK


def _router_body(x_ref, wg_ref, bg_ref, wn_ref, bn_ref, nz_ref,
                 pos0_ref, pos1_ref, p0_ref, p1_ref, be_ref, act_ref,
                 ord_ref, nxte_ref):
    f32 = jnp.float32
    x = x_ref[...]
    logits = jnp.dot(x, wg_ref[...], preferred_element_type=f32) + bg_ref[...]
    nlog = jnp.dot(x, wn_ref[...], preferred_element_type=f32) + bn_ref[...]
    sp = jnp.maximum(nlog, 0.0) + jnp.log1p(jnp.exp(-jnp.abs(nlog)))
    noisy = logits + nz_ref[...] * sp

    idx = lax.broadcasted_iota(jnp.int32, (T_TOKENS, EPAD), 1)
    v1 = jnp.max(noisy, axis=1, keepdims=True)
    i1 = jnp.min(jnp.where(noisy == v1, idx, EPAD - 1), axis=1, keepdims=True)
    m0 = idx == i1
    masked = jnp.where(m0, -1e30, noisy)
    v2 = jnp.max(masked, axis=1, keepdims=True)
    i2 = jnp.min(jnp.where(masked == v2, idx, EPAD - 1), axis=1, keepdims=True)
    m1 = idx == i2
    d = jnp.exp(v2 - v1)
    p0_ref[...] = 1.0 / (1.0 + d)
    p1_ref[...] = d / (1.0 + d)

    m0f = m0.astype(f32)
    m1f = m1.astype(f32)

    blk = 512
    r = lax.broadcasted_iota(jnp.int32, (blk, blk), 0)
    c = lax.broadcasted_iota(jnp.int32, (blk, blk), 1)
    ls = (r > c).astype(f32)
    carry = jnp.zeros((1, EPAD), f32)
    ranks = []
    for m in (m0f, m1f):
        for b in range(T_TOKENS // blk):
            mb = m[b * blk:(b + 1) * blk]
            s = jnp.dot(ls, mb, preferred_element_type=f32) + carry
            ranks.append(jnp.sum(s * mb, axis=1, keepdims=True))
            carry = carry + jnp.sum(mb, axis=0, keepdims=True)
    rank0 = jnp.concatenate(ranks[:T_TOKENS // blk], axis=0)
    rank1 = jnp.concatenate(ranks[T_TOKENS // blk:], axis=0)

    counts = carry
    pc = jnp.floor((counts + (BM - 1)) / BM) * BM
    rr = lax.broadcasted_iota(jnp.int32, (EPAD, EPAD), 0)
    cc = lax.broadcasted_iota(jnp.int32, (EPAD, EPAD), 1)
    tu = (rr < cc).astype(f32)
    offs = jnp.dot(pc, tu, preferred_element_type=f32)

    pos0 = rank0 + jnp.sum(m0f * offs, axis=1, keepdims=True)
    pos1 = rank1 + jnp.sum(m1f * offs, axis=1, keepdims=True)
    pos0_ref[...] = pos0.astype(jnp.int32)
    pos1_ref[...] = pos1.astype(jnp.int32)

    pend = offs + pc
    bstart = (lax.broadcasted_iota(jnp.int32, (NB, 1), 0) * BM).astype(f32)
    lane_ok = lax.broadcasted_iota(jnp.int32, (1, EPAD), 1) < NUM_EXPERTS
    done = jnp.where(lane_ok & (pend <= bstart), 1, 0)
    nd = jnp.sum(done, axis=1, keepdims=True)
    be_ref[...] = jnp.minimum(nd, NUM_EXPERTS - 1).astype(jnp.int32)
    act_ref[...] = jnp.where(nd < NUM_EXPERTS, 1, 0).astype(jnp.int32)

    nz = lane_ok & (pc > 0.0)
    ord_ref[...] = jnp.sum(jnp.where(nz & (pend <= bstart), 1, 0),
                           axis=1, keepdims=True).astype(jnp.int32)
    unf = nz & (pend > bstart)
    ee = lax.broadcasted_iota(jnp.int32, (NB, EPAD), 1)
    curv = jnp.min(jnp.where(unf, ee, EPAD), axis=1, keepdims=True)
    nxtv = jnp.min(jnp.where(unf & (ee > curv), ee, EPAD), axis=1,
                   keepdims=True)
    nxte = jnp.where(nxtv >= EPAD, jnp.minimum(curv, NUM_EXPERTS - 1), nxtv)
    nxte_ref[...] = jnp.minimum(nxte, NUM_EXPERTS - 1).astype(jnp.int32)


def _run_router(xr, wg_p, bg_p, wn_p, bn_p, noise_p):
    f32 = jnp.float32
    return pl.pallas_call(
        _router_body,
        out_shape=[
            jax.ShapeDtypeStruct((T_TOKENS, 1), jnp.int32),
            jax.ShapeDtypeStruct((T_TOKENS, 1), jnp.int32),
            jax.ShapeDtypeStruct((T_TOKENS, 1), f32),
            jax.ShapeDtypeStruct((T_TOKENS, 1), f32),
            jax.ShapeDtypeStruct((NB, 1), jnp.int32),
            jax.ShapeDtypeStruct((NB, 1), jnp.int32),
            jax.ShapeDtypeStruct((NB, 1), jnp.int32),
            jax.ShapeDtypeStruct((NB, 1), jnp.int32),
        ],
        compiler_params=pltpu.CompilerParams(
            vmem_limit_bytes=100 * 1024 * 1024),
    )(xr, wg_p, bg_p, wn_p, bn_p, noise_p)


def _dispatch_body(x_hbm, idx_hbm, xs_hbm, idx_v, xa_v, sem):
    wid = lax.axis_index("s") * 2 + lax.axis_index("c")
    pltpu.sync_copy(idx_hbm.at[wid], idx_v)
    base = wid * TPW
    for ch in range(NCH):
        pltpu.sync_copy(x_hbm.at[pl.ds(base + ch * CHUNK, CHUNK)], xa_v)
        pltpu.async_copy(xa_v, xs_hbm.at[idx_v.at[ch]], sem).wait()
        pltpu.async_copy(xa_v, xs_hbm.at[idx_v.at[NCH + ch]], sem).wait()


def _run_dispatch(xr, idx_all):
    mesh = plsc.VectorSubcoreMesh(core_axis_name="c", subcore_axis_name="s")
    return pl.kernel(
        _dispatch_body,
        mesh=mesh,
        out_type=jax.ShapeDtypeStruct((NSLOT, N_EMBED), jnp.float32),
        scratch_types=[
            pltpu.VMEM((2 * NCH, CHUNK), jnp.int32),
            pltpu.VMEM((CHUNK, N_EMBED), jnp.float32),
            pltpu.SemaphoreType.DMA,
        ],
    )(xr, idx_all)


KQ = D_FF // 4


def _gemm_body(be_ref, act_ref, ord_ref, nxte_ref, x_ref, b1_ref, b2_ref,
               w1_hbm, w2_hbm, o_ref, w1v, w2v, sem1, sem2):
    f32 = jnp.float32
    b = pl.program_id(0)
    cur = be_ref[b]
    prev = be_ref[jnp.maximum(b - 1, 0)]
    changed = jnp.logical_or(b == 0, cur != prev)
    p = lax.rem(ord_ref[b], 2)

    def w1cp(e, slot):
        return pltpu.make_async_copy(w1_hbm.at[e], w1v.at[slot],
                                     sem1.at[slot])

    @pl.when(b == 0)
    def _():
        w1cp(cur, 0).start()

    @pl.when(jnp.logical_and(changed, act_ref[b] == 1))
    def _():
        for k in range(4):
            pltpu.make_async_copy(w2_hbm.at[cur, pl.ds(k * KQ, KQ)],
                                  w2v.at[pl.ds(k * KQ, KQ)],
                                  sem2.at[k]).start()
        nx = nxte_ref[b]

        @pl.when(p == 0)
        def _():
            w1cp(cur, 0).wait()

            @pl.when(nx != cur)
            def _():
                w1cp(nx, 1).start()

        @pl.when(p == 1)
        def _():
            w1cp(cur, 1).wait()

            @pl.when(nx != cur)
            def _():
                w1cp(nx, 0).start()

    @pl.when(act_ref[b] == 1)
    def _():
        x = x_ref[...]
        h = jnp.dot(x, w1v[p], preferred_element_type=f32) + b1_ref[cur]
        h = jnp.maximum(h, 0.0)

        @pl.when(changed)
        def _():
            for k in range(4):
                pltpu.make_async_copy(w2_hbm.at[cur, pl.ds(k * KQ, KQ)],
                                      w2v.at[pl.ds(k * KQ, KQ)],
                                      sem2.at[k]).wait()

        o_ref[...] = jnp.dot(h, w2v[...], preferred_element_type=f32) \
            + b2_ref[cur]


def _run_gemm(xs, w1, b1r, w2, b2r, be, act, ordv, nxte):
    f32 = jnp.float32
    grid_spec = pltpu.PrefetchScalarGridSpec(
        num_scalar_prefetch=4,
        grid=(NB,),
        in_specs=[
            pl.BlockSpec((BM, N_EMBED), lambda b, *_: (b, 0)),
            pl.BlockSpec((NUM_EXPERTS, 1, D_FF), lambda b, *_: (0, 0, 0)),
            pl.BlockSpec((NUM_EXPERTS, 1, N_EMBED), lambda b, *_: (0, 0, 0)),
            pl.BlockSpec(memory_space=pl.ANY),
            pl.BlockSpec(memory_space=pl.ANY),
        ],
        out_specs=pl.BlockSpec((BM, N_EMBED), lambda b, *_: (b, 0)),
        scratch_shapes=[
            pltpu.VMEM((2, N_EMBED, D_FF), f32),
            pltpu.VMEM((D_FF, N_EMBED), f32),
            pltpu.SemaphoreType.DMA((2,)),
            pltpu.SemaphoreType.DMA((4,)),
        ],
    )
    return pl.pallas_call(
        _gemm_body,
        grid_spec=grid_spec,
        out_shape=jax.ShapeDtypeStruct((NSLOT, N_EMBED), jnp.float32),
        compiler_params=pltpu.CompilerParams(
            vmem_limit_bytes=64 * 1024 * 1024),
    )(be, act, ordv, nxte, xs, b1r, b2r, w1, w2)


def _combine_body(ys_hbm, idx_hbm, p_hbm, out_hbm,
                  idx_v, p_v, buf0, buf1, sem):
    wid = lax.axis_index("s") * 2 + lax.axis_index("c")
    pltpu.sync_copy(idx_hbm.at[wid], idx_v)
    pltpu.sync_copy(p_hbm.at[wid], p_v)
    base = wid * TPW
    lanes = N_EMBED // 16
    for ch in range(NCH):
        pltpu.async_copy(ys_hbm.at[idx_v.at[ch]], buf0, sem).wait()
        pltpu.async_copy(ys_hbm.at[idx_v.at[NCH + ch]], buf1, sem).wait()
        for j in range(CHUNK):
            half = pl.ds((j // 16) * 16, 16)
            pa = p_v[ch, half]
            pb = p_v[NCH + ch, half]
            sel = jnp.full((16, 1), j % 16, jnp.int32)
            dnums = lax.GatherDimensionNumbers(
                offset_dims=(), collapsed_slice_dims=(0,),
                start_index_map=(0,))
            w0 = lax.gather(pa, sel, dnums, (1,),
                            mode=lax.GatherScatterMode.PROMISE_IN_BOUNDS)
            w1 = lax.gather(pb, sel, dnums, (1,),
                            mode=lax.GatherScatterMode.PROMISE_IN_BOUNDS)

            def body(v, _):
                sl = pl.ds(v * 16, 16)
                buf0[j, sl] = buf0[j, sl] * w0 + buf1[j, sl] * w1
                return _

            lax.fori_loop(0, lanes, body, 0, unroll=4)
        pltpu.sync_copy(buf0, out_hbm.at[pl.ds(base + ch * CHUNK, CHUNK)])


def _run_combine(ys, idx_all, p_all):
    mesh = plsc.VectorSubcoreMesh(core_axis_name="c", subcore_axis_name="s")
    return pl.kernel(
        _combine_body,
        mesh=mesh,
        out_type=jax.ShapeDtypeStruct((T_TOKENS, N_EMBED), jnp.float32),
        scratch_types=[
            pltpu.VMEM((2 * NCH, CHUNK), jnp.int32),
            pltpu.VMEM((2 * NCH, CHUNK), jnp.float32),
            pltpu.VMEM((CHUNK, N_EMBED), jnp.float32),
            pltpu.VMEM((CHUNK, N_EMBED), jnp.float32),
            pltpu.SemaphoreType.DMA,
        ],
    )(ys, idx_all, p_all)


def kernel(x, Wg, bg, Wn, bn, W1, b1, W2, b2):
    f32 = jnp.float32
    B, S, D = x.shape
    xr = x.reshape(B * S, D)

    wg_p = jnp.pad(Wg, ((0, 0), (0, EPAD - NUM_EXPERTS)))
    wn_p = jnp.pad(Wn, ((0, 0), (0, EPAD - NUM_EXPERTS)))
    bg_p = jnp.pad(bg, (0, EPAD - NUM_EXPERTS),
                   constant_values=-1e30).reshape(1, EPAD)
    bn_p = jnp.pad(bn, (0, EPAD - NUM_EXPERTS)).reshape(1, EPAD)
    noise = jax.random.normal(jax.random.key(42), (B * S, NUM_EXPERTS),
                              dtype=f32)
    noise_p = jnp.pad(noise, ((0, 0), (0, EPAD - NUM_EXPERTS)))

    (pos0, pos1, p0, p1, be, act,
     ordv, nxte) = _run_router(xr, wg_p, bg_p, wn_p, bn_p, noise_p)

    idx_all = jnp.concatenate(
        [pos0.reshape(NW, NCH, CHUNK), pos1.reshape(NW, NCH, CHUNK)], axis=1)
    p_all = jnp.concatenate(
        [p0.reshape(NW, NCH, CHUNK), p1.reshape(NW, NCH, CHUNK)], axis=1)

    xs = _run_dispatch(xr, idx_all)
    ys = _run_gemm(xs, W1, b1.reshape(NUM_EXPERTS, 1, D_FF), W2,
                   b2.reshape(NUM_EXPERTS, 1, N_EMBED), be.reshape(NB),
                   act.reshape(NB), ordv.reshape(NB), nxte.reshape(NB))
    out = _run_combine(ys, idx_all, p_all)
    return out.reshape(B, S, D)
